# baseline (speedup 1.0000x reference)
import functools

import jax
import jax.numpy as jnp
from jax import lax
from jax.experimental import pallas as pl
from jax.experimental.pallas import tpu as pltpu
from jax.experimental.pallas import tpu_sc as plsc

_LANES = 16
_CHUNK = 128


@functools.partial(jax.jit, static_argnums=(2, 3))
def _ppd_loss(flat_logits, targets, n, c):
    info = plsc.get_sparse_core_info()
    nc, ns = info.num_cores, info.num_subcores
    nw = nc * ns
    b_per_w = n // nw
    n_chunks = b_per_w // _CHUNK
    sub = _CHUNK // _LANES

    mesh = plsc.VectorSubcoreMesh(core_axis_name="c", subcore_axis_name="s")

    @functools.partial(
        pl.kernel,
        mesh=mesh,
        out_type=[
            jax.ShapeDtypeStruct((nw, _LANES), jnp.float32),
            jax.ShapeDtypeStruct((nw, _LANES), jnp.float32),
        ],
        scratch_types=[
            pltpu.VMEM((b_per_w,), jnp.int32),
            pltpu.VMEM((n_chunks, _CHUNK), jnp.int32),
            pltpu.VMEM((n_chunks, _CHUNK), jnp.float32),
            pltpu.VMEM((_LANES,), jnp.float32),
            pltpu.VMEM((_LANES,), jnp.float32),
            pltpu.SemaphoreType.DMA,
        ],
    )
    def sc_kernel(
        logits_hbm, tgt_hbm, loss_out, cnt_out, tgt_v, idx_v, gath_v, lsum_v, csum_v, sem
    ):
        wid = lax.axis_index("s") * nc + lax.axis_index("c")
        base = wid * b_per_w
        pltpu.sync_copy(tgt_hbm.at[pl.ds(base, b_per_w)], tgt_v)

        lane = lax.iota(jnp.int32, _LANES)

        def idx_body(j, carry):
            for u in range(sub):
                off = j * _CHUNK + u * _LANES
                t = tgt_v[pl.ds(off, _LANES)]
                tcl = jnp.clip(t, 0, c - 1)
                row = base + off + lane
                idx_v[j, pl.ds(u * _LANES, _LANES)] = row * c + tcl
            return carry

        lax.fori_loop(0, n_chunks, idx_body, 0)

        def fire(j, carry):
            pltpu.async_copy(logits_hbm.at[idx_v.at[j]], gath_v.at[j], sem)
            return carry

        lax.fori_loop(0, n_chunks, fire, 0)

        def drain(j, carry):
            pltpu.make_async_copy(
                logits_hbm.at[idx_v.at[j]], gath_v.at[j], sem
            ).wait()
            return carry

        lax.fori_loop(0, n_chunks, drain, 0)

        def loss_body(j, carry):
            acc, cnt = carry
            for u in range(sub):
                off = j * _CHUNK + u * _LANES
                t = tgt_v[pl.ds(off, _LANES)]
                x = gath_v[j, pl.ds(u * _LANES, _LANES)]
                valid = t != -1
                d = 1.0 - x
                acc = acc + jnp.where(valid, d * d, 0.0)
                cnt = cnt + jnp.where(valid, 1.0, 0.0)
            return acc, cnt

        zero = jnp.zeros((_LANES,), jnp.float32)
        acc, cnt = lax.fori_loop(0, n_chunks, loss_body, (zero, zero))
        lsum_v[...] = acc
        csum_v[...] = cnt
        pltpu.sync_copy(lsum_v, loss_out.at[wid])
        pltpu.sync_copy(csum_v, cnt_out.at[wid])

    loss_p, cnt_p = sc_kernel(flat_logits, targets)

    def reduce_body(loss_ref, cnt_ref, out_ref):
        s = jnp.sum(loss_ref[...])
        nvalid = jnp.maximum(jnp.sum(cnt_ref[...]), 1.0)
        out_ref[...] = jnp.broadcast_to(s / nvalid, (1, 1))

    total = pl.pallas_call(
        reduce_body,
        out_shape=jax.ShapeDtypeStruct((1, 1), jnp.float32),
    )(loss_p, cnt_p)
    return total[0, 0]


def kernel(contrast_logits, contrast_target):
    n, c = contrast_logits.shape
    flat = contrast_logits.reshape(-1)
    tgt = contrast_target.astype(jnp.int32)
    return _ppd_loss(flat, tgt, n, c)
